# Initial kernel scaffold; baseline (speedup 1.0000x reference)
import jax
import jax.numpy as jnp
from jax.experimental import pallas as pl


def kernel(x_agent, x_poi, edge_index_spatial, edge_index_interacts, edge_attr_interacts, Wl11, Wr11, a11, b11, Wl12, Wr12, We12, a12, b12, Wl21, Wr21, a21, b21, Wl22, Wr22, We22, a22, b22, Wa, ba, Wp, bp):
    raise NotImplementedError("write your pallas kernel here")


# speedup vs baseline: 2.2366x; 2.2366x over previous
import functools

import jax
import jax.numpy as jnp
from jax.experimental import pallas as pl

N_AGENT = 10000
N_POI = 10000


def _mm_body(x_ref, w_ref, o_ref):
    o_ref[...] = jnp.dot(x_ref[...], w_ref[...], preferred_element_type=jnp.float32)


def _mm(x, w, bm=2048):
    M, K = x.shape
    _, N = w.shape
    return pl.pallas_call(
        _mm_body,
        grid=(pl.cdiv(M, bm),),
        in_specs=[
            pl.BlockSpec((bm, K), lambda i: (i, 0)),
            pl.BlockSpec((K, N), lambda i: (0, 0)),
        ],
        out_specs=pl.BlockSpec((bm, N), lambda i: (i, 0)),
        out_shape=jax.ShapeDtypeStruct((M, N), jnp.float32),
    )(x, w)


def _gatv2_edges(xl, xr, ei, att, b, n_dst, eat=None):
    src, dst = ei[0], ei[1]
    m = xl[src] + xr[dst]
    if eat is not None:
        m = m + eat
    m = jnp.where(m > 0, m, 0.2 * m)
    e = m @ att
    ex = jnp.exp(e)
    den = jax.ops.segment_sum(ex, dst, num_segments=n_dst)
    num = jax.ops.segment_sum(ex[:, None] * xl[src], dst, num_segments=n_dst)
    return num / (den[:, None] + 1e-16) + b


def kernel(x_agent, x_poi, edge_index_spatial, edge_index_interacts, edge_attr_interacts,
           Wl11, Wr11, a11, b11, Wl12, Wr12, We12, a12, b12,
           Wl21, Wr21, a21, b21, Wl22, Wr22, We22, a22, b22,
           Wa, ba, Wp, bp):
    xl11 = _mm(x_agent, Wl11)
    xr11 = _mm(x_agent, Wr11)
    xl12 = _mm(x_agent, Wl12)
    xr12 = _mm(x_poi, Wr12)
    eat12 = _mm(edge_attr_interacts, We12)
    h_a = _gatv2_edges(xl11, xr11, edge_index_spatial, a11, b11, N_AGENT)
    h_p = _gatv2_edges(xl12, xr12, edge_index_interacts, a12, b12, N_POI, eat12)
    h_a = jnp.maximum(h_a, 0.0)
    h_p = jnp.maximum(h_p, 0.0)
    xl21 = _mm(h_a, Wl21)
    xr21 = _mm(h_a, Wr21)
    xl22 = _mm(h_a, Wl22)
    xr22 = _mm(h_p, Wr22)
    eat22 = _mm(edge_attr_interacts, We22)
    h_a2 = _gatv2_edges(xl21, xr21, edge_index_spatial, a21, b21, N_AGENT)
    h_p2 = _gatv2_edges(xl22, xr22, edge_index_interacts, a22, b22, N_POI, eat22)
    h_a2 = jnp.maximum(h_a2, 0.0)
    h_p2 = jnp.maximum(h_p2, 0.0)
    out_a = _mm(h_a2, Wa) + ba
    out_p = _mm(h_p2, Wp) + bp
    return out_a, out_p
